# baseline (device time: 62777 ns/iter reference)
import jax
import jax.numpy as jnp
from jax import lax
from jax.experimental import pallas as pl
from jax.experimental.pallas import tpu as pltpu

B = 16
NB = 128
BS = 16
H = 16
D = 64
P_LOCAL = 128
T_LOCAL = P_LOCAL * BS
HB = H * B
HD = H * D
SCALE = D ** -0.5
NEG = -1e30


def kernel(Q, K, V, bt, lens):
    q3 = (Q.reshape(B, H, D) * SCALE).transpose(1, 0, 2)
    qexp = (q3[:, :, None, :] * jnp.eye(H, dtype=q3.dtype)[:, None, :, None])
    qexp = qexp.reshape(HB, HD).astype(jnp.bfloat16)
    k2 = K.reshape(T_LOCAL, HD)
    v2 = V.reshape(T_LOCAL, HD)
    lens2 = lens.reshape(B, 1)

    def body(q_ref, k_ref, v_ref, bt_ref, lens_ref, out_ref,
             k_vmem, v_vmem, o_send, st_send, o_recv, st_recv,
             k_sem, v_sem, send_sems, recv_sems):
        my_x = lax.axis_index("x")
        my_y = lax.axis_index("y")
        peer = (1 - my_x, my_y)

        k_dma = pltpu.make_async_copy(k_ref, k_vmem, k_sem)
        v_dma = pltpu.make_async_copy(v_ref, v_vmem, v_sem)
        k_dma.start()
        v_dma.start()

        barrier = pltpu.get_barrier_semaphore()
        pl.semaphore_signal(barrier, inc=1, device_id=peer,
                            device_id_type=pl.DeviceIdType.MESH)
        pl.semaphore_wait(barrier, 1)

        x_off = my_x * P_LOCAL
        bt_arr = bt_ref[...]
        lens_arr = lens_ref[...]
        slot = lax.broadcasted_iota(jnp.int32, (B, NB, P_LOCAL), 1)
        page = lax.broadcasted_iota(jnp.int32, (B, NB, P_LOCAL), 2)
        hit = (bt_arr[:, :, None] == page + x_off) & (
            slot < lens_arr[:, :, None])
        w = jnp.sum(hit.astype(jnp.float32), axis=1)
        logw = jnp.where(w > 0, jnp.log(w), NEG).astype(jnp.bfloat16)

        r_iota = lax.broadcasted_iota(jnp.int32, (HB, B), 0)
        c_iota = lax.broadcasted_iota(jnp.int32, (HB, B), 1)
        tile2 = (r_iota % B == c_iota).astype(jnp.bfloat16)
        a_bias = lax.dot_general(
            tile2, logw, (((1,), (0,)), ((), ())),
            preferred_element_type=jnp.float32,
        ).astype(jnp.bfloat16)
        t_iota = lax.broadcasted_iota(jnp.int32, (T_LOCAL, P_LOCAL), 0)
        p_iota = lax.broadcasted_iota(jnp.int32, (T_LOCAL, P_LOCAL), 1)
        g2 = (t_iota // BS == p_iota).astype(jnp.bfloat16)
        lw = lax.dot_general(
            a_bias, g2, (((1,), (1,)), ((), ())),
            preferred_element_type=jnp.float32,
        )

        k_dma.wait()
        k_bf = k_vmem[...].astype(jnp.bfloat16)
        s = lax.dot_general(
            q_ref[...], k_bf, (((1,), (1,)), ((), ())),
            preferred_element_type=jnp.float32,
        ) + lw
        m = jnp.max(s, axis=1, keepdims=True)
        p_un = jnp.exp((s - m).astype(jnp.bfloat16))
        l = jnp.sum(p_un, axis=1, keepdims=True,
                    dtype=jnp.float32)
        v_dma.wait()
        v_bf = v_vmem[...].astype(jnp.bfloat16)
        o2 = lax.dot_general(
            p_un, v_bf, (((1,), (0,)), ((), ())),
            preferred_element_type=jnp.float32,
        )

        for h in range(H):
            o_send[h] = o2[h * B:(h + 1) * B, h * D:(h + 1) * D]
        st_send[:, 0:1] = m
        st_send[:, 1:2] = l

        rdma_o = pltpu.make_async_remote_copy(
            src_ref=o_send, dst_ref=o_recv,
            send_sem=send_sems.at[0], recv_sem=recv_sems.at[0],
            device_id=peer, device_id_type=pl.DeviceIdType.MESH,
        )
        rdma_st = pltpu.make_async_remote_copy(
            src_ref=st_send, dst_ref=st_recv,
            send_sem=send_sems.at[1], recv_sem=recv_sems.at[1],
            device_id=peer, device_id_type=pl.DeviceIdType.MESH,
        )
        rdma_o.start()
        rdma_st.start()
        rdma_o.wait()
        rdma_st.wait()

        m_loc = st_send[:, 0:1]
        l_loc = st_send[:, 1:2]
        m_p = st_recv[:, 0:1]
        l_p = st_recv[:, 1:2]
        m_new = jnp.maximum(m_loc, m_p)
        a = jnp.exp(m_loc - m_new)
        c = jnp.exp(m_p - m_new)
        l_new = l_loc * a + l_p * c
        a3 = a.reshape(H, B, 1)
        c3 = c.reshape(H, B, 1)
        l3 = l_new.reshape(H, B, 1)
        out_ref[...] = (o_send[...] * a3 + o_recv[...] * c3) / l3

    out = pl.pallas_call(
        body,
        out_shape=jax.ShapeDtypeStruct((H, B, D), jnp.float32),
        in_specs=[
            pl.BlockSpec(memory_space=pltpu.VMEM),
            pl.BlockSpec(memory_space=pl.ANY),
            pl.BlockSpec(memory_space=pl.ANY),
            pl.BlockSpec(memory_space=pltpu.VMEM),
            pl.BlockSpec(memory_space=pltpu.VMEM),
        ],
        out_specs=pl.BlockSpec(memory_space=pltpu.VMEM),
        scratch_shapes=[
            pltpu.VMEM((T_LOCAL, HD), jnp.float32),
            pltpu.VMEM((T_LOCAL, HD), jnp.float32),
            pltpu.VMEM((H, B, D), jnp.float32),
            pltpu.VMEM((HB, 2), jnp.float32),
            pltpu.VMEM((H, B, D), jnp.float32),
            pltpu.VMEM((HB, 2), jnp.float32),
            pltpu.SemaphoreType.DMA,
            pltpu.SemaphoreType.DMA,
            pltpu.SemaphoreType.DMA((2,)),
            pltpu.SemaphoreType.DMA((2,)),
        ],
        compiler_params=pltpu.CompilerParams(collective_id=0),
    )(qexp, k2, v2, bt, lens2)

    return out.swapaxes(0, 1).reshape(B, 1, H, D)


# device time: 35249 ns/iter; 1.7810x vs baseline; 1.7810x over previous
import jax
import jax.numpy as jnp
from jax import lax
from jax.experimental import pallas as pl
from jax.experimental.pallas import tpu as pltpu

B = 16
NB = 128
BS = 16
H = 16
D = 64
P_LOCAL = 128
T_LOCAL = P_LOCAL * BS
SCALE = D ** -0.5
NEG = -1e30


def kernel(Q, K, V, bt, lens):
    q = (Q.reshape(B, H, D) * SCALE).astype(jnp.bfloat16).swapaxes(0, 1)
    k = K.reshape(T_LOCAL, H, D).astype(jnp.bfloat16).swapaxes(0, 1)
    v = V.reshape(T_LOCAL, H, D).astype(jnp.bfloat16).swapaxes(0, 1)
    lens2 = lens.reshape(B, 1)

    def body(q_ref, k_ref, v_ref, bt_ref, lens_ref, out_ref,
             logw_ref, o_send, st_send, o_recv, st_recv,
             send_sems, recv_sems):
        h = pl.program_id(0)
        my_x = lax.axis_index("x")
        my_y = lax.axis_index("y")
        peer = (1 - my_x, my_y)

        @pl.when(h == 0)
        def _prologue():
            barrier = pltpu.get_barrier_semaphore()
            pl.semaphore_signal(barrier, inc=1, device_id=peer,
                                device_id_type=pl.DeviceIdType.MESH)
            pl.semaphore_wait(barrier, 1)

            x_off = my_x * P_LOCAL
            bt_arr = bt_ref[...]
            lens_arr = lens_ref[...]
            slot = lax.broadcasted_iota(jnp.int32, (B, NB, P_LOCAL), 1)
            page = lax.broadcasted_iota(jnp.int32, (B, NB, P_LOCAL), 2)
            hit = (bt_arr[:, :, None] == page + x_off) & (
                slot < lens_arr[:, :, None])
            w = jnp.sum(hit.astype(jnp.float32), axis=1)
            logw = jnp.where(w > 0, jnp.log(w), NEG).astype(jnp.bfloat16)

            tpage = lax.broadcasted_iota(
                jnp.int32, (P_LOCAL, T_LOCAL), 1) // BS
            prow = lax.broadcasted_iota(jnp.int32, (P_LOCAL, T_LOCAL), 0)
            expand = (tpage == prow).astype(jnp.bfloat16)
            logw_ref[...] = lax.dot_general(
                logw, expand,
                (((1,), (0,)), ((), ())),
                preferred_element_type=jnp.float32,
            )

        qh = q_ref[...]
        kh = k_ref[...]
        s = lax.dot_general(
            qh, kh, (((1,), (1,)), ((), ())),
            preferred_element_type=jnp.float32,
        ) + logw_ref[...]
        m_h = jnp.max(s, axis=1, keepdims=True)
        p_un = jnp.exp((s - m_h).astype(jnp.bfloat16))
        l_h = jnp.sum(p_un, axis=1, keepdims=True,
                      dtype=jnp.float32)
        vh = v_ref[...]
        o_h = lax.dot_general(
            p_un, vh, (((1,), (0,)), ((), ())),
            preferred_element_type=jnp.float32,
        )

        o_send[pl.ds(h, 1)] = o_h[None, :, :]
        st_send[pl.ds(h, 1)] = jnp.concatenate([m_h, l_h], axis=1)[None]

        @pl.when(h == H - 1)
        def _epilogue():
            rdma_o = pltpu.make_async_remote_copy(
                src_ref=o_send, dst_ref=o_recv,
                send_sem=send_sems.at[0], recv_sem=recv_sems.at[0],
                device_id=peer, device_id_type=pl.DeviceIdType.MESH,
            )
            rdma_st = pltpu.make_async_remote_copy(
                src_ref=st_send, dst_ref=st_recv,
                send_sem=send_sems.at[1], recv_sem=recv_sems.at[1],
                device_id=peer, device_id_type=pl.DeviceIdType.MESH,
            )
            rdma_o.start()
            rdma_st.start()
            rdma_o.wait()
            rdma_st.wait()

            m_loc = st_send[:, :, 0:1]
            l_loc = st_send[:, :, 1:2]
            o_loc = o_send[...]
            m_p = st_recv[:, :, 0:1]
            l_p = st_recv[:, :, 1:2]
            o_p = o_recv[...]
            m_new = jnp.maximum(m_loc, m_p)
            a = jnp.exp(m_loc - m_new)
            c = jnp.exp(m_p - m_new)
            l_new = l_loc * a + l_p * c
            out_ref[...] = (o_loc * a + o_p * c) / l_new

    out = pl.pallas_call(
        body,
        grid=(H,),
        out_shape=jax.ShapeDtypeStruct((H, B, D), jnp.float32),
        in_specs=[
            pl.BlockSpec((None, B, D), lambda h: (h, 0, 0)),
            pl.BlockSpec((None, T_LOCAL, D), lambda h: (h, 0, 0)),
            pl.BlockSpec((None, T_LOCAL, D), lambda h: (h, 0, 0)),
            pl.BlockSpec((B, NB), lambda h: (0, 0)),
            pl.BlockSpec((B, 1), lambda h: (0, 0)),
        ],
        out_specs=pl.BlockSpec((H, B, D), lambda h: (0, 0, 0)),
        scratch_shapes=[
            pltpu.VMEM((B, T_LOCAL), jnp.float32),
            pltpu.VMEM((H, B, D), jnp.float32),
            pltpu.VMEM((H, B, 2), jnp.float32),
            pltpu.VMEM((H, B, D), jnp.float32),
            pltpu.VMEM((H, B, 2), jnp.float32),
            pltpu.SemaphoreType.DMA((2,)),
            pltpu.SemaphoreType.DMA((2,)),
        ],
        compiler_params=pltpu.CompilerParams(collective_id=0),
    )(q, k, v, bt, lens2)

    return out.swapaxes(0, 1).reshape(B, 1, H, D)
